# baseline (speedup 1.0000x reference)
import functools

import jax
import jax.numpy as jnp
from jax import lax
from jax.experimental import pallas as pl
from jax.experimental.pallas import tpu as pltpu
from jax.experimental.pallas import tpu_sc as plsc

_ALPHA = 0.05
_EPS = 1e-8

_NC = 2
_NS = 16
_NW = _NC * _NS


def _stage_masks(n, lj, lk):
    i = lax.broadcasted_iota(jnp.int32, (n, 1), 0)
    bitj = ((i >> lj) & 1) == 1
    take_min = ((i >> lj) & 1) == ((i >> lk) & 1)
    return bitj, take_min


def _partner(x, j, bitj):
    n = x.shape[0]
    return jnp.where(bitj, pltpu.roll(x, j, 0), pltpu.roll(x, n - j, 0))


def _stages(n):
    ln = n.bit_length() - 1
    for lk in range(1, ln + 1):
        for lj in range(lk - 1, -1, -1):
            yield lj, lk


def _sort1_body(old_ref, hsrc_ref):
    x = old_ref[...]
    n = x.shape[0]
    for lj, lk in _stages(n):
        bitj, tm = _stage_masks(n, lj, lk)
        p = _partner(x, 1 << lj, bitj)
        x = jnp.where((x < p) == tm, x, p)
    hsrc_ref[...] = x


def _sort23_body(aligned_ref, hsrc_ref, new_ref, out_ref):
    a = _ALPHA
    k = aligned_ref[...]
    n, c = k.shape
    v = lax.broadcasted_iota(jnp.int32, (n, c), 0)
    for lj, lk in _stages(n):
        j = 1 << lj
        bitj, tm = _stage_masks(n, lj, lk)
        pk = _partner(k, j, bitj)
        pv = _partner(v, j, bitj)
        keep = ((k < pk) | ((k == pk) & (v < pv))) == tm
        k = jnp.where(keep, k, pk)
        v = jnp.where(keep, v, pv)
    s = hsrc_ref[...]
    for lj, lk in _stages(n):
        j = 1 << lj
        bitj, tm = _stage_masks(n, lj, lk)
        pv = _partner(v, j, bitj)
        ps = _partner(s, j, bitj)
        keep = (v < pv) == tm
        v = jnp.where(keep, v, pv)
        s = jnp.where(keep, s, ps)
    out_ref[...] = ((1.0 - a) * new_ref[...]
                    + (a - a * a) * aligned_ref[...]
                    + (a * a) * s)


def _norm_body(x_ref, y_ref):
    x = x_ref[...]
    nrm = jnp.sqrt(jnp.sum(x * x, axis=1, keepdims=True))
    y_ref[...] = x / jnp.maximum(nrm, _EPS)


def _argmax_body(old_ref, newn_ref, idx_ref):
    sim = lax.dot_general(old_ref[...], newn_ref[...],
                          (((1,), (1,)), ((), ())),
                          preferred_element_type=jnp.float32)
    m = jnp.max(sim, axis=1, keepdims=True)
    col = lax.broadcasted_iota(jnp.int32, sim.shape, 1)
    cand = jnp.where(sim == m, col, sim.shape[1])
    idx_ref[...] = jnp.min(cand, axis=1, keepdims=True)


def _normalize(x):
    n, d = x.shape
    blk = min(n, 512)
    return pl.pallas_call(
        _norm_body,
        grid=(n // blk,),
        in_specs=[pl.BlockSpec((blk, d), lambda i: (i, 0))],
        out_specs=pl.BlockSpec((blk, d), lambda i: (i, 0)),
        out_shape=jax.ShapeDtypeStruct((n, d), jnp.float32),
    )(x)


def _top1(hidden_old, new_n):
    n, d = hidden_old.shape
    m = new_n.shape[0]
    blk = 256
    idx = pl.pallas_call(
        _argmax_body,
        grid=(n // blk,),
        in_specs=[pl.BlockSpec((blk, d), lambda i: (i, 0)),
                  pl.BlockSpec((m, d), lambda i: (0, 0))],
        out_specs=pl.BlockSpec((blk, 1), lambda i: (i, 0)),
        out_shape=jax.ShapeDtypeStruct((n, 1), jnp.int32),
    )(hidden_old, new_n)
    return idx.reshape(n)


def _sort_cols(x, blk=128):
    n, d = x.shape
    return pl.pallas_call(
        _sort1_body,
        grid=(d // blk,),
        in_specs=[pl.BlockSpec((n, blk), lambda i: (0, i))],
        out_specs=pl.BlockSpec((n, blk), lambda i: (0, i)),
        out_shape=jax.ShapeDtypeStruct((n, d), jnp.float32),
    )(x)


def _ot_update(aligned, h_src, hidden_new, blk=128):
    n, d = aligned.shape
    return pl.pallas_call(
        _sort23_body,
        grid=(d // blk,),
        in_specs=[pl.BlockSpec((n, blk), lambda i: (0, i))] * 3,
        out_specs=pl.BlockSpec((n, blk), lambda i: (0, i)),
        out_shape=jax.ShapeDtypeStruct((n, d), jnp.float32),
    )(aligned, h_src, hidden_new)


def _gather_rows(table, idx):
    n, d = table.shape
    b = idx.shape[0]
    bpw = b // _NW
    mesh = plsc.VectorSubcoreMesh(core_axis_name="c", subcore_axis_name="s")

    @functools.partial(
        pl.kernel, mesh=mesh,
        out_type=jax.ShapeDtypeStruct((b, d), jnp.float32),
        scratch_types=[
            pltpu.VMEM((bpw,), jnp.int32),
            pltpu.VMEM((bpw, d), jnp.float32),
            pltpu.SemaphoreType.DMA,
        ],
    )
    def k(table_hbm, idx_hbm, out_hbm, idx_v, rows_v, sem):
        wid = lax.axis_index("s") * _NC + lax.axis_index("c")
        base = wid * bpw
        pltpu.sync_copy(idx_hbm.at[pl.ds(base, bpw)], idx_v)
        pltpu.async_copy(table_hbm.at[idx_v], rows_v, sem).wait()
        pltpu.sync_copy(rows_v, out_hbm.at[pl.ds(base, bpw)])

    return k(table, idx)


def kernel(hidden_old, hidden_new):
    new_n = _normalize(hidden_new)
    idx = _top1(hidden_old, new_n)
    aligned = _gather_rows(hidden_new, idx)
    h_src = _sort_cols(hidden_old)
    return _ot_update(aligned, h_src, hidden_new)


# speedup vs baseline: 1.9296x; 1.9296x over previous
import functools

import numpy as np
import jax
import jax.numpy as jnp
from jax import lax
from jax.experimental import pallas as pl
from jax.experimental.pallas import tpu as pltpu
from jax.experimental.pallas import tpu_sc as plsc

_ALPHA = 0.05
_EPS = 1e-8

_NC = 2
_NS = 16
_NW = _NC * _NS


def _stages(n):
    ln = n.bit_length() - 1
    for lk in range(1, ln + 1):
        for lj in range(lk - 1, -1, -1):
            yield lj, lk


def _roll_masks(n, lj, lk):
    i = lax.broadcasted_iota(jnp.int32, (n, 1), 0)
    bitj = ((i >> lj) & 1) == 1
    take_min = (((i >> lj) ^ (i >> lk)) & 1) == 0
    return bitj, take_min


def _asc_mask(m, lj, lk):
    bm = lax.broadcasted_iota(jnp.int32, (m, 1, 1), 0)
    return ((bm >> (lk - lj - 1)) & 1) == 0


def _stage_val(x, lj, lk):
    n, c = x.shape
    j = 1 << lj
    if lj >= 3:
        m = n // (2 * j)
        x3 = x.reshape(m, 2, j, c)
        ah, bh = x3[:, 0], x3[:, 1]
        mn = jnp.minimum(ah, bh)
        mx = jnp.maximum(ah, bh)
        asc = _asc_mask(m, lj, lk)
        na = jnp.where(asc, mn, mx)
        nb = jnp.where(asc, mx, mn)
        return jnp.stack([na, nb], axis=1).reshape(n, c)
    bitj, tm = _roll_masks(n, lj, lk)
    p = jnp.where(bitj, pltpu.roll(x, j, 0), pltpu.roll(x, n - j, 0))
    return jnp.where((x < p) == tm, x, p)


def _sortable(f):
    b = lax.bitcast_convert_type(f, jnp.int32)
    return b ^ ((b >> 31) & jnp.int32(0x7FFFFFFF))


def _sort_old_body(old_ref, k1_ref):
    n = old_ref.shape[0]
    k1 = _sortable(old_ref[...])
    for lj, lk in _stages(n):
        k1 = _stage_val(k1, lj, lk)
    k1_ref[...] = k1


def _sort_aligned_body(aligned_ref, k2_ref):
    n = aligned_ref.shape[0]
    rb = n.bit_length() - 1
    rows = lax.broadcasted_iota(jnp.int32, (n, 1), 0)
    k2 = (_sortable(aligned_ref[...]) & jnp.int32(~((1 << rb) - 1))) | rows
    for lj, lk in _stages(n):
        k2 = _stage_val(k2, lj, lk)
    k2_ref[...] = k2


def _scatter_combine_body(k1_ref, k2_ref, aligned_ref, new_ref, out_ref):
    a = _ALPHA
    n = k1_ref.shape[0]
    rb = n.bit_length() - 1
    pb = 32 - rb
    idx_tgt = k2_ref[...] & jnp.int32((1 << rb) - 1)
    pay = (k1_ref[...] >> rb) & jnp.int32((1 << pb) - 1)
    k3 = ((idx_tgt - jnp.int32(n // 2)) << pb) | pay
    for lj, lk in _stages(n):
        k3 = _stage_val(k3, lj, lk)
    sb = (k3 & jnp.int32((1 << pb) - 1)) << rb
    s = lax.bitcast_convert_type(sb ^ ((sb >> 31) & jnp.int32(0x7FFFFFFF)),
                                 jnp.float32)
    out_ref[...] = ((1.0 - a) * new_ref[...]
                    + (a - a * a) * aligned_ref[...]
                    + (a * a) * s)


def _norm_body(x_ref, y_ref):
    x = x_ref[...]
    nrm = jnp.sqrt(jnp.sum(x * x, axis=1, keepdims=True))
    y_ref[...] = x / jnp.maximum(nrm, _EPS)


def _argmax_body(old_ref, newn_ref, idx_ref):
    sim = lax.dot_general(old_ref[...], newn_ref[...],
                          (((1,), (1,)), ((), ())),
                          preferred_element_type=jnp.float32)
    m = jnp.max(sim, axis=1, keepdims=True)
    col = lax.broadcasted_iota(jnp.int32, sim.shape, 1)
    cand = jnp.where(sim == m, col, sim.shape[1])
    idx_ref[...] = jnp.min(cand, axis=1, keepdims=True)


def _normalize(x):
    n, d = x.shape
    blk = min(n, 512)
    return pl.pallas_call(
        _norm_body,
        grid=(n // blk,),
        in_specs=[pl.BlockSpec((blk, d), lambda i: (i, 0))],
        out_specs=pl.BlockSpec((blk, d), lambda i: (i, 0)),
        out_shape=jax.ShapeDtypeStruct((n, d), jnp.float32),
    )(x)


def _top1(hidden_old, new_n):
    n, d = hidden_old.shape
    m = new_n.shape[0]
    blk = 256
    idx = pl.pallas_call(
        _argmax_body,
        grid=(n // blk,),
        in_specs=[pl.BlockSpec((blk, d), lambda i: (i, 0)),
                  pl.BlockSpec((m, d), lambda i: (0, 0))],
        out_specs=pl.BlockSpec((blk, 1), lambda i: (i, 0)),
        out_shape=jax.ShapeDtypeStruct((n, 1), jnp.int32),
    )(hidden_old, new_n)
    return idx.reshape(n)


def _colspec(n, blk):
    return pl.BlockSpec((n, blk), lambda i: (0, i))


def _ot_update(hidden_old, aligned, hidden_new, blk=128):
    n, d = aligned.shape
    k1 = pl.pallas_call(
        _sort_old_body,
        grid=(d // blk,),
        in_specs=[_colspec(n, blk)],
        out_specs=_colspec(n, blk),
        out_shape=jax.ShapeDtypeStruct((n, d), jnp.int32),
    )(hidden_old)
    k2 = pl.pallas_call(
        _sort_aligned_body,
        grid=(d // blk,),
        in_specs=[_colspec(n, blk)],
        out_specs=_colspec(n, blk),
        out_shape=jax.ShapeDtypeStruct((n, d), jnp.int32),
    )(aligned)
    return pl.pallas_call(
        _scatter_combine_body,
        grid=(d // blk,),
        in_specs=[_colspec(n, blk)] * 4,
        out_specs=_colspec(n, blk),
        out_shape=jax.ShapeDtypeStruct((n, d), jnp.float32),
    )(k1, k2, aligned, hidden_new)


def _gather_rows(table, idx):
    n, d = table.shape
    b = idx.shape[0]
    bpw = b // _NW
    mesh = plsc.VectorSubcoreMesh(core_axis_name="c", subcore_axis_name="s")

    @functools.partial(
        pl.kernel, mesh=mesh,
        out_type=jax.ShapeDtypeStruct((b, d), jnp.float32),
        scratch_types=[
            pltpu.VMEM((bpw,), jnp.int32),
            pltpu.VMEM((bpw, d), jnp.float32),
            pltpu.SemaphoreType.DMA,
        ],
    )
    def k(table_hbm, idx_hbm, out_hbm, idx_v, rows_v, sem):
        wid = lax.axis_index("s") * _NC + lax.axis_index("c")
        base = wid * bpw
        pltpu.sync_copy(idx_hbm.at[pl.ds(base, bpw)], idx_v)
        pltpu.async_copy(table_hbm.at[idx_v], rows_v, sem).wait()
        pltpu.sync_copy(rows_v, out_hbm.at[pl.ds(base, bpw)])

    return k(table, idx)


def kernel(hidden_old, hidden_new):
    new_n = _normalize(hidden_new)
    idx = _top1(hidden_old, new_n)
    aligned = _gather_rows(hidden_new, idx)
    return _ot_update(hidden_old, aligned, hidden_new)
